# baseline (speedup 1.0000x reference)
import functools

import jax
import jax.numpy as jnp
from jax import lax
from jax.experimental import pallas as pl
from jax.experimental.pallas import tpu as pltpu
from jax.experimental.pallas import tpu_sc as plsc

D_MODEL = 64
N_BATCH = 4096
N_TIME = 200
LANES = 128
NUM_WORKERS = 32
VOCAB_ROWS = 1000000

TC_VBLK = 2048


def _t2_body(tt_ref, out_ref):
    out_ref[:, 0:D_MODEL] = jnp.swapaxes(tt_ref[...], 0, 1)


def _make_t2(table_t):
    grid = (VOCAB_ROWS + TC_VBLK - 1) // TC_VBLK
    return pl.pallas_call(
        _t2_body,
        grid=(grid,),
        in_specs=[pl.BlockSpec((D_MODEL, TC_VBLK), lambda i: (0, i))],
        out_specs=pl.BlockSpec((TC_VBLK, LANES), lambda i: (i, 0)),
        out_shape=jax.ShapeDtypeStruct((VOCAB_ROWS, LANES), jnp.float32),
    )(table_t)


@functools.partial(
    pl.kernel,
    mesh=plsc.VectorSubcoreMesh(core_axis_name="c", subcore_axis_name="s"),
    out_type=jax.ShapeDtypeStruct(
        (N_TIME, D_MODEL // 8, N_BATCH // LANES, 8, LANES), jnp.float32),
    scratch_types=[
        pltpu.VMEM((N_TIME * LANES,), jnp.int32),
        pltpu.VMEM((2, LANES, LANES), jnp.float32),
        pltpu.VMEM((2, 8, 8, LANES + 1), jnp.float32),
        pltpu.SemaphoreType.DMA,
        pltpu.SemaphoreType.DMA,
        pltpu.SemaphoreType.DMA,
        pltpu.SemaphoreType.DMA,
        pltpu.SemaphoreType.DMA,
    ],
    compiler_params=pltpu.CompilerParams(needs_layout_passes=False),
)
def _gather(t2_hbm, xw_hbm, out_hbm,
            x_v, g_v, d_v, xsem, gsem0, gsem1, wsem0, wsem1):
    wid = lax.axis_index("s") * 2 + lax.axis_index("c")
    gsems = (gsem0, gsem1)
    wsems = (wsem0, wsem1)
    n_slabs = N_TIME

    pltpu.make_async_copy(xw_hbm.at[wid], x_v, xsem).start()
    pltpu.make_async_copy(xw_hbm.at[wid], x_v, xsem).wait()

    def g_dma(k, s):
        return pltpu.make_async_copy(
            t2_hbm.at[x_v.at[pl.ds(k * LANES, LANES)]],
            g_v.at[s], gsems[s])

    def w_dma(k, s):
        return pltpu.make_async_copy(
            d_v.at[s, :, :, pl.ds(0, LANES)],
            out_hbm.at[k, :, wid], wsems[s])

    iota = lax.iota(jnp.int32, 16)
    rows_hi = [(iota + d0 * 16) >> 3 for d0 in range(D_MODEL // 16)]
    rows_lo = [(iota + d0 * 16) & 7 for d0 in range(D_MODEL // 16)]
    one = (iota & 0) + 1

    def transpose_slab(s):
        eight = (iota & 0) + 8

        def tok8(blk, base):
            m0 = blk * 8
            for j in range(8):
                col = base + j
                for d0 in range(D_MODEL // 16):
                    vals = g_v[s, m0 + j, pl.ds(d0 * 16, 16)]
                    plsc.store_scatter(
                        d_v.at[s], [rows_hi[d0], rows_lo[d0], col], vals)
            return base + eight

        lax.fori_loop(0, LANES // 8, tok8, iota & 0)

    g_dma(0, 0).start()

    def body(j, carry):
        k0 = j * 2
        for b in range(2):
            k = k0 + b
            s = b

            @pl.when(k + 1 < n_slabs)
            def _():
                g_dma(k + 1, 1 - s).start()

            g_dma(k, s).wait()

            @pl.when(k >= 2)
            def _():
                w_dma(k - 2, s).wait()

            transpose_slab(s)
            w_dma(k, s).start()
        return carry

    lax.fori_loop(0, n_slabs // 2, body, 0)
    w_dma(n_slabs - 2, 0).wait()
    w_dma(n_slabs - 1, 1).wait()


def kernel(x, table):
    t2 = _make_t2(table.T)
    xw = (
        x.astype(jnp.int32)
        .T.reshape(N_TIME, NUM_WORKERS, LANES)
        .transpose(1, 0, 2)
        .reshape(NUM_WORKERS, N_TIME * LANES)
    )
    out5 = _gather(t2, xw)
    return out5.transpose(2, 4, 0, 1, 3).reshape(N_BATCH, N_TIME, D_MODEL)


# speedup vs baseline: 1.2979x; 1.2979x over previous
import functools

import jax
import jax.numpy as jnp
from jax import lax
from jax.experimental import pallas as pl
from jax.experimental.pallas import tpu as pltpu
from jax.experimental.pallas import tpu_sc as plsc

D_MODEL = 64
N_BATCH = 4096
N_TIME = 200
LANES = 128
NUM_WORKERS = 32
VOCAB_ROWS = 1000000

TC_VBLK = 2048


def _eye(n):
    r = lax.broadcasted_iota(jnp.int32, (n, n), 0)
    c = lax.broadcasted_iota(jnp.int32, (n, n), 1)
    return (r == c).astype(jnp.float32)


def _t2_body(tt_ref, out_ref):
    out_ref[:, 0:D_MODEL] = lax.dot_general(
        tt_ref[...], _eye(D_MODEL), (((0,), (0,)), ((), ())),
        preferred_element_type=jnp.float32)


def _make_t2(table_t):
    grid = (VOCAB_ROWS + TC_VBLK - 1) // TC_VBLK
    return pl.pallas_call(
        _t2_body,
        grid=(grid,),
        in_specs=[pl.BlockSpec((D_MODEL, TC_VBLK), lambda i: (0, i))],
        out_specs=pl.BlockSpec((TC_VBLK, LANES), lambda i: (i, 0)),
        out_shape=jax.ShapeDtypeStruct((VOCAB_ROWS, LANES), jnp.float32),
    )(table_t)


def _out_body(lin_ref, out_ref):
    eye = _eye(LANES)
    for c in range(N_BATCH // LANES):
        seg = lin_ref[0, pl.ds(c * LANES, LANES), 0:D_MODEL]
        t = lax.dot_general(
            seg, eye, (((0,), (0,)), ((), ())),
            preferred_element_type=jnp.float32)
        out_ref[0, :, c, :, :] = t.reshape(8, 8, LANES)


def _make_out5(out_lin):
    return pl.pallas_call(
        _out_body,
        grid=(N_TIME,),
        in_specs=[pl.BlockSpec(
            (1, N_BATCH, LANES), lambda t: (t, 0, 0))],
        out_specs=pl.BlockSpec(
            (1, D_MODEL // 8, N_BATCH // LANES, 8, LANES),
            lambda t: (t, 0, 0, 0, 0)),
        out_shape=jax.ShapeDtypeStruct(
            (N_TIME, D_MODEL // 8, N_BATCH // LANES, 8, LANES),
            jnp.float32),
    )(out_lin)


@functools.partial(
    pl.kernel,
    mesh=plsc.VectorSubcoreMesh(core_axis_name="c", subcore_axis_name="s"),
    out_type=jax.ShapeDtypeStruct((N_TIME, N_BATCH, LANES), jnp.float32),
    scratch_types=[
        pltpu.VMEM((N_TIME * LANES,), jnp.int32),
        pltpu.VMEM((2, LANES, LANES), jnp.float32),
        pltpu.SemaphoreType.DMA,
        pltpu.SemaphoreType.DMA,
        pltpu.SemaphoreType.DMA,
        pltpu.SemaphoreType.DMA,
        pltpu.SemaphoreType.DMA,
    ],
    compiler_params=pltpu.CompilerParams(needs_layout_passes=False),
)
def _gather(t2_hbm, xw_hbm, out_hbm,
            x_v, g_v, xsem, gsem0, gsem1, wsem0, wsem1):
    wid = lax.axis_index("s") * 2 + lax.axis_index("c")
    gsems = (gsem0, gsem1)
    wsems = (wsem0, wsem1)

    pltpu.make_async_copy(xw_hbm.at[wid], x_v, xsem).start()
    pltpu.make_async_copy(xw_hbm.at[wid], x_v, xsem).wait()

    def g_dma(t, s):
        return pltpu.make_async_copy(
            t2_hbm.at[x_v.at[pl.ds(t * LANES, LANES)]],
            g_v.at[s], gsems[s])

    def w_dma(t, s):
        return pltpu.make_async_copy(
            g_v.at[s],
            out_hbm.at[t, pl.ds(wid * LANES, LANES)], wsems[s])

    g_dma(0, 0).start()

    def body(k, carry):
        t0 = k * 2
        for b in range(2):
            t = t0 + b
            s = b

            g_dma(t, s).wait()
            w_dma(t, s).start()

            @pl.when(t >= 1)
            def _():
                w_dma(t - 1, 1 - s).wait()

            @pl.when(t + 1 < N_TIME)
            def _():
                g_dma(t + 1, 1 - s).start()
        return carry

    lax.fori_loop(0, N_TIME // 2, body, 0)
    w_dma(N_TIME - 1, 1).wait()


def kernel(x, table):
    t2 = _make_t2(table.T)
    xw = (
        x.astype(jnp.int32)
        .T.reshape(N_TIME, NUM_WORKERS, LANES)
        .transpose(1, 0, 2)
        .reshape(NUM_WORKERS, N_TIME * LANES)
    )
    out_lin = _gather(t2, xw)
    out5 = _make_out5(out_lin)
    return out5.transpose(2, 4, 0, 1, 3).reshape(N_BATCH, N_TIME, D_MODEL)


# speedup vs baseline: 1.6036x; 1.2355x over previous
import functools

import jax
import jax.numpy as jnp
from jax import lax
from jax.experimental import pallas as pl
from jax.experimental.pallas import tpu as pltpu
from jax.experimental.pallas import tpu_sc as plsc

D_MODEL = 64
N_BATCH = 4096
N_TIME = 200
LANES = 128
NUM_WORKERS = 32
VOCAB_ROWS = 1000000

TC_VBLK = 8192


def _eye(n):
    r = lax.broadcasted_iota(jnp.int32, (n, n), 0)
    c = lax.broadcasted_iota(jnp.int32, (n, n), 1)
    return (r == c).astype(jnp.float32)


def _t2_body(tt_ref, out_ref):
    out_ref[:, 0:D_MODEL] = lax.dot_general(
        tt_ref[...], _eye(D_MODEL), (((0,), (0,)), ((), ())),
        preferred_element_type=jnp.float32)


def _make_t2(table_t):
    grid = (VOCAB_ROWS + TC_VBLK - 1) // TC_VBLK
    return pl.pallas_call(
        _t2_body,
        grid=(grid,),
        in_specs=[pl.BlockSpec((D_MODEL, TC_VBLK), lambda i: (0, i))],
        out_specs=pl.BlockSpec((TC_VBLK, LANES), lambda i: (i, 0)),
        out_shape=jax.ShapeDtypeStruct((VOCAB_ROWS, LANES), jnp.float32),
    )(table_t)


def _out_body(lin_ref, out_ref):
    eye = _eye(LANES)
    for c in range(N_BATCH // LANES):
        seg = lin_ref[0, pl.ds(c * LANES, LANES), 0:D_MODEL]
        t = lax.dot_general(
            seg, eye, (((0,), (0,)), ((), ())),
            preferred_element_type=jnp.float32)
        out_ref[0, :, c, :, :] = t.reshape(8, 8, LANES)


def _make_out5(out_lin):
    return pl.pallas_call(
        _out_body,
        grid=(N_TIME,),
        in_specs=[pl.BlockSpec(
            (1, N_BATCH, LANES), lambda t: (t, 0, 0))],
        out_specs=pl.BlockSpec(
            (1, D_MODEL // 8, N_BATCH // LANES, 8, LANES),
            lambda t: (t, 0, 0, 0, 0)),
        out_shape=jax.ShapeDtypeStruct(
            (N_TIME, D_MODEL // 8, N_BATCH // LANES, 8, LANES),
            jnp.float32),
    )(out_lin)


@functools.partial(
    pl.kernel,
    mesh=plsc.VectorSubcoreMesh(core_axis_name="c", subcore_axis_name="s"),
    out_type=jax.ShapeDtypeStruct((N_TIME, N_BATCH, LANES), jnp.float32),
    scratch_types=[
        pltpu.VMEM((N_TIME * LANES,), jnp.int32),
        pltpu.VMEM((2, LANES, LANES), jnp.float32),
        pltpu.SemaphoreType.DMA,
        pltpu.SemaphoreType.DMA,
        pltpu.SemaphoreType.DMA,
        pltpu.SemaphoreType.DMA,
        pltpu.SemaphoreType.DMA,
    ],
    compiler_params=pltpu.CompilerParams(needs_layout_passes=False),
)
def _gather(t2_hbm, xw_hbm, out_hbm,
            x_v, g_v, xsem, gsem0, gsem1, wsem0, wsem1):
    wid = lax.axis_index("s") * 2 + lax.axis_index("c")
    gsems = (gsem0, gsem1)
    wsems = (wsem0, wsem1)

    pltpu.make_async_copy(xw_hbm.at[wid], x_v, xsem).start()
    pltpu.make_async_copy(xw_hbm.at[wid], x_v, xsem).wait()

    def g_dma(t, s):
        return pltpu.make_async_copy(
            t2_hbm.at[x_v.at[pl.ds(t * LANES, LANES)]],
            g_v.at[s], gsems[s])

    def w_dma(t, s):
        return pltpu.make_async_copy(
            g_v.at[s],
            out_hbm.at[t, pl.ds(wid * LANES, LANES)], wsems[s])

    g_dma(0, 0).start()

    def body(k, carry):
        t0 = k * 2
        for b in range(2):
            t = t0 + b
            s = b

            g_dma(t, s).wait()
            w_dma(t, s).start()

            @pl.when(t >= 1)
            def _():
                w_dma(t - 1, 1 - s).wait()

            @pl.when(t + 1 < N_TIME)
            def _():
                g_dma(t + 1, 1 - s).start()
        return carry

    lax.fori_loop(0, N_TIME // 2, body, 0)
    w_dma(N_TIME - 1, 1).wait()


def kernel(x, table):
    t2 = _make_t2(table.T)
    xw = (
        x.astype(jnp.int32)
        .T.reshape(N_TIME, NUM_WORKERS, LANES)
        .transpose(1, 0, 2)
        .reshape(NUM_WORKERS, N_TIME * LANES)
    )
    out_lin = _gather(t2, xw)
    out5 = _make_out5(out_lin)
    return out5.transpose(2, 4, 0, 1, 3).reshape(N_BATCH, N_TIME, D_MODEL)


# speedup vs baseline: 1.7915x; 1.1172x over previous
import functools

import jax
import jax.numpy as jnp
from jax import lax
from jax.experimental import pallas as pl
from jax.experimental.pallas import tpu as pltpu
from jax.experimental.pallas import tpu_sc as plsc

D_MODEL = 64
N_BATCH = 4096
N_TIME = 200
LANES = 128
NUM_WORKERS = 32
VOCAB_ROWS = 1000000

TC_VBLK = 16384


def _eye(n):
    r = lax.broadcasted_iota(jnp.int32, (n, n), 0)
    c = lax.broadcasted_iota(jnp.int32, (n, n), 1)
    return (r == c).astype(jnp.float32)


def _t2_body(tt_ref, out_ref):
    out_ref[:, 0:D_MODEL] = lax.dot_general(
        tt_ref[...], _eye(D_MODEL), (((0,), (0,)), ((), ())),
        preferred_element_type=jnp.float32)


def _make_t2(table_t):
    grid = (VOCAB_ROWS + TC_VBLK - 1) // TC_VBLK
    return pl.pallas_call(
        _t2_body,
        grid=(grid,),
        in_specs=[pl.BlockSpec((D_MODEL, TC_VBLK), lambda i: (0, i))],
        out_specs=pl.BlockSpec((TC_VBLK, LANES), lambda i: (i, 0)),
        out_shape=jax.ShapeDtypeStruct((VOCAB_ROWS, LANES), jnp.float32),
    )(table_t)


T_BLK = 4


def _out_body(lin_ref, out_ref):
    eye = _eye(LANES)
    for tt in range(T_BLK):
        for c in range(N_BATCH // LANES):
            seg = lin_ref[tt, pl.ds(c * LANES, LANES), 0:D_MODEL]
            t = lax.dot_general(
                seg, eye, (((0,), (0,)), ((), ())),
                preferred_element_type=jnp.float32)
            out_ref[tt, :, c, :, :] = t.reshape(8, 8, LANES)


def _make_out5(out_lin):
    return pl.pallas_call(
        _out_body,
        grid=(N_TIME // T_BLK,),
        in_specs=[pl.BlockSpec(
            (T_BLK, N_BATCH, LANES), lambda t: (t, 0, 0))],
        out_specs=pl.BlockSpec(
            (T_BLK, D_MODEL // 8, N_BATCH // LANES, 8, LANES),
            lambda t: (t, 0, 0, 0, 0)),
        out_shape=jax.ShapeDtypeStruct(
            (N_TIME, D_MODEL // 8, N_BATCH // LANES, 8, LANES),
            jnp.float32),
    )(out_lin)


@functools.partial(
    pl.kernel,
    mesh=plsc.VectorSubcoreMesh(core_axis_name="c", subcore_axis_name="s"),
    out_type=jax.ShapeDtypeStruct((N_TIME, N_BATCH, LANES), jnp.float32),
    scratch_types=[
        pltpu.VMEM((N_TIME * LANES,), jnp.int32),
        pltpu.VMEM((2, LANES, LANES), jnp.float32),
        pltpu.SemaphoreType.DMA,
        pltpu.SemaphoreType.DMA,
        pltpu.SemaphoreType.DMA,
        pltpu.SemaphoreType.DMA,
        pltpu.SemaphoreType.DMA,
    ],
    compiler_params=pltpu.CompilerParams(needs_layout_passes=False),
)
def _gather(t2_hbm, xw_hbm, out_hbm,
            x_v, g_v, xsem, gsem0, gsem1, wsem0, wsem1):
    wid = lax.axis_index("s") * 2 + lax.axis_index("c")
    gsems = (gsem0, gsem1)
    wsems = (wsem0, wsem1)

    pltpu.make_async_copy(xw_hbm.at[wid], x_v, xsem).start()
    pltpu.make_async_copy(xw_hbm.at[wid], x_v, xsem).wait()

    def g_dma(t, s):
        return pltpu.make_async_copy(
            t2_hbm.at[x_v.at[pl.ds(t * LANES, LANES)]],
            g_v.at[s], gsems[s])

    def w_dma(t, s):
        return pltpu.make_async_copy(
            g_v.at[s],
            out_hbm.at[t, pl.ds(wid * LANES, LANES)], wsems[s])

    g_dma(0, 0).start()

    def body(k, carry):
        t0 = k * 2
        for b in range(2):
            t = t0 + b
            s = b

            g_dma(t, s).wait()
            w_dma(t, s).start()

            @pl.when(t >= 1)
            def _():
                w_dma(t - 1, 1 - s).wait()

            @pl.when(t + 1 < N_TIME)
            def _():
                g_dma(t + 1, 1 - s).start()
        return carry

    lax.fori_loop(0, N_TIME // 2, body, 0)
    w_dma(N_TIME - 1, 1).wait()


def kernel(x, table):
    t2 = _make_t2(table.T)
    xw = (
        x.astype(jnp.int32)
        .T.reshape(N_TIME, NUM_WORKERS, LANES)
        .transpose(1, 0, 2)
        .reshape(NUM_WORKERS, N_TIME * LANES)
    )
    out_lin = _gather(t2, xw)
    out5 = _make_out5(out_lin)
    return out5.transpose(2, 4, 0, 1, 3).reshape(N_BATCH, N_TIME, D_MODEL)
